# baseline (speedup 1.0000x reference)
import functools

import jax
import jax.numpy as jnp
from jax import lax
from jax.experimental import pallas as pl
from jax.experimental.pallas import tpu as pltpu
from jax.experimental.pallas import tpu_sc as plsc

_N = 10000
_DIM = 128
_E = 320000
_EU = 160000
_NLAYERS = 3
_NC, _NS, _NW = 2, 16, 32
_CH = 128
_KCH = 80
_EPAD = _NW * _KCH * _CH
_KCH_E = 40
_EUPAD = _NW * _KCH_E * _CH
_ACC = 10112
_STRIPE = _ACC // _NS

@functools.lru_cache(maxsize=None)
def _mesh():
    return plsc.VectorSubcoreMesh(core_axis_name="c", subcore_axis_name="s")


def _deg_body(dst_i, ones_h, zeros_h, out, acc, ones_v, didx, ss0, ss1):
    c = lax.axis_index("c")
    s = lax.axis_index("s")
    wid = c * _NS + s
    row0 = s * _STRIPE
    pltpu.sync_copy(zeros_h.at[pl.ds(row0, _STRIPE)], acc.at[pl.ds(row0, _STRIPE)])
    pltpu.sync_copy(ones_h, ones_v)
    pltpu.sync_copy(dst_i.at[pl.ds(wid * _KCH, _KCH)], didx)
    plsc.subcore_barrier()

    def scatter(k, sem):
        return pltpu.async_copy(ones_v, acc.at[didx.at[k]], sem, add=True)

    def wait_s(k, sem):
        pltpu.make_async_copy(ones_v, acc.at[didx.at[k]], sem).wait()

    scatter(0, ss0)
    scatter(1, ss1)

    def step(kk, carry):
        k0 = 2 * kk
        wait_s(k0 - 2, ss0)
        scatter(k0, ss0)
        wait_s(k0 - 1, ss1)
        scatter(k0 + 1, ss1)
        return carry

    lax.fori_loop(1, _KCH // 2, step, 0)
    wait_s(_KCH - 2, ss0)
    wait_s(_KCH - 1, ss1)
    plsc.subcore_barrier()
    pltpu.sync_copy(acc.at[pl.ds(row0, _STRIPE)], out.at[c, pl.ds(row0, _STRIPE)])


@functools.lru_cache(maxsize=None)
def _deg_kernel():
    return pl.kernel(
        _deg_body,
        out_type=jax.ShapeDtypeStruct((_NC, _ACC, _DIM), jnp.float32),
        mesh=_mesh(),
        scratch_types=[
            pltpu.VMEM_SHARED((_ACC, _DIM), jnp.float32),
            pltpu.VMEM((_CH, _DIM), jnp.float32),
            pltpu.VMEM((_KCH, _CH), jnp.int32),
            pltpu.SemaphoreType.DMA,
            pltpu.SemaphoreType.DMA,
        ],
    )


def _agg_body(table, src_i, dst_i, zeros_h, out, acc, sidx, didx,
              rows0, rows1, gs0, gs1, ss0, ss1):
    c = lax.axis_index("c")
    s = lax.axis_index("s")
    wid = c * _NS + s
    row0 = s * _STRIPE
    pltpu.sync_copy(zeros_h.at[pl.ds(row0, _STRIPE)], acc.at[pl.ds(row0, _STRIPE)])
    plsc.subcore_barrier()

    def gather(k, buf, sem):
        return pltpu.async_copy(table.at[sidx.at[k]], buf, sem)

    def scatter(k, buf, sem):
        return pltpu.async_copy(buf, acc.at[didx.at[k]], sem, add=True)

    kh = _KCH // 2
    for half in range(2):
        hb = wid * _KCH + half * kh
        pltpu.sync_copy(src_i.at[pl.ds(hb, kh)], sidx)
        pltpu.sync_copy(dst_i.at[pl.ds(hb, kh)], didx)
        gather(0, rows0, gs0).wait()
        gather(1, rows1, gs1)
        scatter(0, rows0, ss0)

        def step(kk, carry):
            k0 = 2 * kk
            pltpu.make_async_copy(table.at[sidx.at[k0 - 1]], rows1, gs1).wait()
            pltpu.make_async_copy(rows0, acc.at[didx.at[k0 - 2]], ss0).wait()
            gather(k0, rows0, gs0)
            scatter(k0 - 1, rows1, ss1)
            pltpu.make_async_copy(table.at[sidx.at[k0]], rows0, gs0).wait()
            pltpu.make_async_copy(rows1, acc.at[didx.at[k0 - 1]], ss1).wait()
            gather(k0 + 1, rows1, gs1)
            scatter(k0, rows0, ss0)
            return carry

        lax.fori_loop(1, kh // 2, step, 0)
        pltpu.make_async_copy(table.at[sidx.at[kh - 1]], rows1, gs1).wait()
        pltpu.make_async_copy(rows0, acc.at[didx.at[kh - 2]], ss0).wait()
        scatter(kh - 1, rows1, ss1).wait()
    plsc.subcore_barrier()
    pltpu.sync_copy(acc.at[pl.ds(row0, _STRIPE)], out.at[c, pl.ds(row0, _STRIPE)])


@functools.lru_cache(maxsize=None)
def _agg_kernel():
    return pl.kernel(
        _agg_body,
        out_type=jax.ShapeDtypeStruct((_NC, _ACC, _DIM), jnp.float32),
        mesh=_mesh(),
        scratch_types=[
            pltpu.VMEM_SHARED((_ACC, _DIM), jnp.float32),
            pltpu.VMEM((_KCH // 2, _CH), jnp.int32),
            pltpu.VMEM((_KCH // 2, _CH), jnp.int32),
            pltpu.VMEM((_CH, _DIM), jnp.float32),
            pltpu.VMEM((_CH, _DIM), jnp.float32),
            pltpu.SemaphoreType.DMA,
            pltpu.SemaphoreType.DMA,
            pltpu.SemaphoreType.DMA,
            pltpu.SemaphoreType.DMA,
        ],
    )


def _egather_body(ta, tb, src_i, dst_i, outa, outb, sidx, didx,
                  ba0, bb0, ba1, bb1,
                  gsa0, gsb0, gsa1, gsb1, wsa0, wsb0, wsa1, wsb1):
    c = lax.axis_index("c")
    s = lax.axis_index("s")
    wid = c * _NS + s
    base = wid * _KCH_E * _CH
    pltpu.sync_copy(src_i.at[pl.ds(wid * _KCH_E, _KCH_E)], sidx)
    pltpu.sync_copy(dst_i.at[pl.ds(wid * _KCH_E, _KCH_E)], didx)

    def gath(k, bufa, bufb, sa, sb):
        da = pltpu.async_copy(ta.at[sidx.at[k]], bufa, sa)
        db = pltpu.async_copy(tb.at[didx.at[k]], bufb, sb)
        return da, db

    def wrt(k, bufa, bufb, sa, sb):
        off = base + k * _CH
        da = pltpu.async_copy(bufa, outa.at[pl.ds(off, _CH)], sa)
        db = pltpu.async_copy(bufb, outb.at[pl.ds(off, _CH)], sb)
        return da, db

    def wait_g(k, bufa, bufb, sa, sb):
        pltpu.make_async_copy(ta.at[sidx.at[k]], bufa, sa).wait()
        pltpu.make_async_copy(tb.at[didx.at[k]], bufb, sb).wait()

    def wait_w(k, bufa, bufb, sa, sb):
        off = base + k * _CH
        pltpu.make_async_copy(bufa, outa.at[pl.ds(off, _CH)], sa).wait()
        pltpu.make_async_copy(bufb, outb.at[pl.ds(off, _CH)], sb).wait()

    gath(0, ba0, bb0, gsa0, gsb0)
    wait_g(0, ba0, bb0, gsa0, gsb0)
    gath(1, ba1, bb1, gsa1, gsb1)
    wrt(0, ba0, bb0, wsa0, wsb0)

    def step(kk, carry):
        k0 = 2 * kk
        wait_g(k0 - 1, ba1, bb1, gsa1, gsb1)
        wait_w(k0 - 2, ba0, bb0, wsa0, wsb0)
        gath(k0, ba0, bb0, gsa0, gsb0)
        wrt(k0 - 1, ba1, bb1, wsa1, wsb1)
        wait_g(k0, ba0, bb0, gsa0, gsb0)
        wait_w(k0 - 1, ba1, bb1, wsa1, wsb1)
        gath(k0 + 1, ba1, bb1, gsa1, gsb1)
        wrt(k0, ba0, bb0, wsa0, wsb0)
        return carry

    lax.fori_loop(1, _KCH_E // 2, step, 0)
    wait_g(_KCH_E - 1, ba1, bb1, gsa1, gsb1)
    wait_w(_KCH_E - 2, ba0, bb0, wsa0, wsb0)
    wrt(_KCH_E - 1, ba1, bb1, wsa1, wsb1)
    wait_w(_KCH_E - 1, ba1, bb1, wsa1, wsb1)


@functools.lru_cache(maxsize=None)
def _egather_kernel():
    return pl.kernel(
        _egather_body,
        out_type=(
            jax.ShapeDtypeStruct((_EUPAD, _DIM), jnp.float32),
            jax.ShapeDtypeStruct((_EUPAD, _DIM), jnp.float32),
        ),
        mesh=_mesh(),
        scratch_types=[
            pltpu.VMEM((_KCH_E, _CH), jnp.int32),
            pltpu.VMEM((_KCH_E, _CH), jnp.int32),
            pltpu.VMEM((_CH, _DIM), jnp.float32),
            pltpu.VMEM((_CH, _DIM), jnp.float32),
            pltpu.VMEM((_CH, _DIM), jnp.float32),
            pltpu.VMEM((_CH, _DIM), jnp.float32),
        ] + [pltpu.SemaphoreType.DMA] * 8,
    )


_BR = 2000


def _mm_scale_body(h_ref, w_ref, degp_ref, out_ref):
    dinv = lax.rsqrt(degp_ref[0, :, 0:1] + degp_ref[1, :, 0:1] + 1.0)
    hw = lax.dot_general(h_ref[...], w_ref[...], (((1,), (1,)), ((), ())),
                         preferred_element_type=jnp.float32)
    out_ref[...] = hw * dinv


def _mm_scale(h, w, degp):
    return pl.pallas_call(
        _mm_scale_body,
        grid=(_N // _BR,),
        in_specs=[
            pl.BlockSpec((_BR, _DIM), lambda i: (i, 0)),
            pl.BlockSpec((_DIM, _DIM), lambda i: (0, 0)),
            pl.BlockSpec((_NC, _BR, _DIM), lambda i: (0, i, 0)),
        ],
        out_specs=pl.BlockSpec((_BR, _DIM), lambda i: (i, 0)),
        out_shape=jax.ShapeDtypeStruct((_N, _DIM), jnp.float32),
    )(h, w, degp)


def _ln_res_body(h_ref, hwp_ref, sp_ref, degp_ref, cb_ref, lw_ref, lb_ref,
                 out_ref):
    dinv = lax.rsqrt(degp_ref[0, :, 0:1] + degp_ref[1, :, 0:1] + 1.0)
    seg = sp_ref[0] + sp_ref[1] + hwp_ref[...]
    u = h_ref[...] + dinv * seg + cb_ref[...]
    mu = jnp.mean(u, axis=-1, keepdims=True)
    d = u - mu
    var = jnp.mean(d * d, axis=-1, keepdims=True)
    out_ref[...] = d * lax.rsqrt(var + 1e-5) * lw_ref[...] + lb_ref[...]


def _ln_res(h, hwp, sp, degp, cb, lw, lb):
    return pl.pallas_call(
        _ln_res_body,
        grid=(_N // _BR,),
        in_specs=[
            pl.BlockSpec((_BR, _DIM), lambda i: (i, 0)),
            pl.BlockSpec((_BR, _DIM), lambda i: (i, 0)),
            pl.BlockSpec((_NC, _BR, _DIM), lambda i: (0, i, 0)),
            pl.BlockSpec((_NC, _BR, _DIM), lambda i: (0, i, 0)),
            pl.BlockSpec((1, _DIM), lambda i: (0, 0)),
            pl.BlockSpec((1, _DIM), lambda i: (0, 0)),
            pl.BlockSpec((1, _DIM), lambda i: (0, 0)),
        ],
        out_specs=pl.BlockSpec((_BR, _DIM), lambda i: (i, 0)),
        out_shape=jax.ShapeDtypeStruct((_N, _DIM), jnp.float32),
    )(h, hwp, sp, degp, cb, lw, lb)


def _ab_body(h_ref, wa_ref, wb_ref, outa_ref, outb_ref):
    h = h_ref[...]
    outa_ref[...] = jnp.dot(h, wa_ref[...], preferred_element_type=jnp.float32)
    outb_ref[...] = jnp.dot(h, wb_ref[...], preferred_element_type=jnp.float32)


def _ab_proj(h, wa, wb):
    return pl.pallas_call(
        _ab_body,
        grid=(_N // _BR,),
        in_specs=[
            pl.BlockSpec((_BR, _DIM), lambda i: (i, 0)),
            pl.BlockSpec((_DIM, _DIM), lambda i: (0, 0)),
            pl.BlockSpec((_DIM, _DIM), lambda i: (0, 0)),
        ],
        out_specs=(
            pl.BlockSpec((_BR, _DIM), lambda i: (i, 0)),
            pl.BlockSpec((_BR, _DIM), lambda i: (i, 0)),
        ),
        out_shape=(
            jax.ShapeDtypeStruct((_N, _DIM), jnp.float32),
            jax.ShapeDtypeStruct((_N, _DIM), jnp.float32),
        ),
    )(h, wa, wb)


_BE = 2000


def _emlp_body(ga_ref, gb_ref, ef_ref, wc_ref, w2_ref, b1_ref, b2_ref,
               out_ref):
    t = (ga_ref[...] + gb_ref[...]
         + jnp.dot(ef_ref[...], wc_ref[...], preferred_element_type=jnp.float32)
         + b1_ref[...])
    hid = jnp.maximum(t, 0.0)
    out_ref[...] = (jnp.dot(hid, w2_ref[...], preferred_element_type=jnp.float32)
                    + b2_ref[...])


def _emlp(ga, gb, ef, wc, w2, b1, b2):
    return pl.pallas_call(
        _emlp_body,
        grid=(_EU // _BE,),
        in_specs=[
            pl.BlockSpec((_BE, _DIM), lambda i: (i, 0)),
            pl.BlockSpec((_BE, _DIM), lambda i: (i, 0)),
            pl.BlockSpec((_BE, 16), lambda i: (i, 0)),
            pl.BlockSpec((16, _DIM), lambda i: (0, 0)),
            pl.BlockSpec((_DIM, _DIM), lambda i: (0, 0)),
            pl.BlockSpec((1, _DIM), lambda i: (0, 0)),
            pl.BlockSpec((1, _DIM), lambda i: (0, 0)),
        ],
        out_specs=pl.BlockSpec((_BE, _DIM), lambda i: (i, 0)),
        out_shape=jax.ShapeDtypeStruct((_EU, _DIM), jnp.float32),
    )(ga, gb, ef, wc, w2, b1, b2)


def kernel(x, edge_index, edge_features, convW, convB, lnW, lnB, eW1, eB1,
           eW2, eB2):
    src_all = edge_index[0]
    dst_all = edge_index[1]
    pad_e = _EPAD - _E
    src_pad = jnp.concatenate([src_all, jnp.zeros((pad_e,), jnp.int32)])
    src_pad = src_pad.reshape(_NW * _KCH, _CH)
    dst_pad = jnp.concatenate([dst_all, jnp.full((pad_e,), _N, jnp.int32)])
    dst_pad = dst_pad.reshape(_NW * _KCH, _CH)

    zeros128 = jnp.zeros((_ACC, _DIM), jnp.float32)
    ones128 = jnp.ones((_CH, _DIM), jnp.float32)

    degp = _deg_kernel()(dst_pad, ones128, zeros128)

    h = x
    for l in range(_NLAYERS):
        hwp = _mm_scale(h, convW[l], degp)
        sp = _agg_kernel()(hwp, src_pad, dst_pad, zeros128)
        h = _ln_res(h, hwp, sp, degp, convB[l].reshape(1, _DIM),
                    lnW[l].reshape(1, _DIM), lnB[l].reshape(1, _DIM))

    srcu = edge_index[0, 0::2]
    dstu = edge_index[1, 0::2]
    pad_u = _EUPAD - _EU
    srcu_pad = jnp.concatenate([srcu, jnp.zeros((pad_u,), jnp.int32)])
    srcu_pad = srcu_pad.reshape(_NW * _KCH_E, _CH)
    dstu_pad = jnp.concatenate([dstu, jnp.zeros((pad_u,), jnp.int32)])
    dstu_pad = dstu_pad.reshape(_NW * _KCH_E, _CH)

    e_w1t = eW1.T
    wa = e_w1t[:_DIM]
    wb = e_w1t[_DIM:2 * _DIM]
    wc = e_w1t[2 * _DIM:]
    a, b = _ab_proj(h, wa, wb)
    ga, gb = _egather_kernel()(a, b, srcu_pad, dstu_pad)
    edge_emb = _emlp(ga, gb, edge_features, wc, eW2.T,
                     eB1.reshape(1, _DIM), eB2.reshape(1, _DIM))
    return (h, edge_emb)


# speedup vs baseline: 2.5284x; 2.5284x over previous
import functools

import jax
import jax.numpy as jnp
from jax import lax
from jax.experimental import pallas as pl
from jax.experimental.pallas import tpu as pltpu
from jax.experimental.pallas import tpu_sc as plsc

_N = 10000
_DIM = 128
_E = 320000
_EU = 160000
_NLAYERS = 3
_NC, _NS, _NW = 2, 16, 32
_CH = 128
_KCH = 80
_EPAD = _NW * _KCH * _CH
_KCH_E = 40
_EUPAD = _NW * _KCH_E * _CH
_ACC = 10112
_STRIPE = _ACC // _NS

@functools.lru_cache(maxsize=None)
def _mesh():
    return plsc.VectorSubcoreMesh(core_axis_name="c", subcore_axis_name="s")


def _deg_body(dst_i, ones_h, zeros_h, out, acc, ones_v, didx, ss0, ss1):
    c = lax.axis_index("c")
    s = lax.axis_index("s")
    wid = c * _NS + s
    row0 = s * _STRIPE
    pltpu.sync_copy(zeros_h.at[pl.ds(row0, _STRIPE)], acc.at[pl.ds(row0, _STRIPE)])
    pltpu.sync_copy(ones_h, ones_v)
    pltpu.sync_copy(dst_i.at[pl.ds(wid * _KCH, _KCH)], didx)
    plsc.subcore_barrier()

    def scatter(k, sem):
        return pltpu.async_copy(ones_v, acc.at[didx.at[k]], sem, add=True)

    def wait_s(k, sem):
        pltpu.make_async_copy(ones_v, acc.at[didx.at[k]], sem).wait()

    scatter(0, ss0)
    scatter(1, ss1)

    def step(kk, carry):
        k0 = 2 * kk
        wait_s(k0 - 2, ss0)
        scatter(k0, ss0)
        wait_s(k0 - 1, ss1)
        scatter(k0 + 1, ss1)
        return carry

    lax.fori_loop(1, _KCH // 2, step, 0)
    wait_s(_KCH - 2, ss0)
    wait_s(_KCH - 1, ss1)
    plsc.subcore_barrier()
    pltpu.sync_copy(acc.at[pl.ds(row0, _STRIPE)], out.at[c, pl.ds(row0, _STRIPE)])


@functools.lru_cache(maxsize=None)
def _deg_kernel():
    return pl.kernel(
        _deg_body,
        out_type=jax.ShapeDtypeStruct((_NC, _ACC, _DIM), jnp.float32),
        mesh=_mesh(),
        scratch_types=[
            pltpu.VMEM_SHARED((_ACC, _DIM), jnp.float32),
            pltpu.VMEM((_CH, _DIM), jnp.float32),
            pltpu.VMEM((_KCH, _CH), jnp.int32),
            pltpu.SemaphoreType.DMA,
            pltpu.SemaphoreType.DMA,
        ],
    )


def _agg_body(table, src_i, dst_i, zeros_h, out, acc, sidx, didx,
              rows0, rows1, gs0, gs1, ss0, ss1):
    c = lax.axis_index("c")
    s = lax.axis_index("s")
    wid = c * _NS + s
    row0 = s * _STRIPE
    pltpu.sync_copy(zeros_h.at[pl.ds(row0, _STRIPE)], acc.at[pl.ds(row0, _STRIPE)])
    plsc.subcore_barrier()

    def gather(k, buf, sem):
        return pltpu.async_copy(table.at[sidx.at[k]], buf, sem)

    def scatter(k, buf, sem):
        return pltpu.async_copy(buf, acc.at[didx.at[k]], sem, add=True)

    kh = _KCH // 2
    for half in range(2):
        hb = wid * _KCH + half * kh
        pltpu.sync_copy(src_i.at[pl.ds(hb, kh)], sidx)
        pltpu.sync_copy(dst_i.at[pl.ds(hb, kh)], didx)
        gather(0, rows0, gs0).wait()
        gather(1, rows1, gs1)
        scatter(0, rows0, ss0)

        def step(kk, carry):
            k0 = 2 * kk
            pltpu.make_async_copy(table.at[sidx.at[k0 - 1]], rows1, gs1).wait()
            pltpu.make_async_copy(rows0, acc.at[didx.at[k0 - 2]], ss0).wait()
            gather(k0, rows0, gs0)
            scatter(k0 - 1, rows1, ss1)
            pltpu.make_async_copy(table.at[sidx.at[k0]], rows0, gs0).wait()
            pltpu.make_async_copy(rows1, acc.at[didx.at[k0 - 1]], ss1).wait()
            gather(k0 + 1, rows1, gs1)
            scatter(k0, rows0, ss0)
            return carry

        lax.fori_loop(1, kh // 2, step, 0)
        pltpu.make_async_copy(table.at[sidx.at[kh - 1]], rows1, gs1).wait()
        pltpu.make_async_copy(rows0, acc.at[didx.at[kh - 2]], ss0).wait()
        scatter(kh - 1, rows1, ss1).wait()
    plsc.subcore_barrier()
    pltpu.sync_copy(acc.at[pl.ds(row0, _STRIPE)], out.at[c, pl.ds(row0, _STRIPE)])


@functools.lru_cache(maxsize=None)
def _agg_kernel():
    return pl.kernel(
        _agg_body,
        out_type=jax.ShapeDtypeStruct((_NC, _ACC, _DIM), jnp.float32),
        mesh=_mesh(),
        scratch_types=[
            pltpu.VMEM_SHARED((_ACC, _DIM), jnp.float32),
            pltpu.VMEM((_KCH // 2, _CH), jnp.int32),
            pltpu.VMEM((_KCH // 2, _CH), jnp.int32),
            pltpu.VMEM((_CH, _DIM), jnp.float32),
            pltpu.VMEM((_CH, _DIM), jnp.float32),
            pltpu.SemaphoreType.DMA,
            pltpu.SemaphoreType.DMA,
            pltpu.SemaphoreType.DMA,
            pltpu.SemaphoreType.DMA,
        ],
    )


def _egather_body(ta, tb, src_i, dst_i, outa, outb, sidx, didx,
                  ba0, bb0, ba1, bb1,
                  gsa0, gsb0, gsa1, gsb1, wsa0, wsb0, wsa1, wsb1):
    c = lax.axis_index("c")
    s = lax.axis_index("s")
    wid = c * _NS + s
    base = wid * _KCH_E * _CH
    pltpu.sync_copy(src_i.at[pl.ds(wid * _KCH_E, _KCH_E)], sidx)
    pltpu.sync_copy(dst_i.at[pl.ds(wid * _KCH_E, _KCH_E)], didx)

    def gath(k, bufa, bufb, sa, sb):
        da = pltpu.async_copy(ta.at[sidx.at[k]], bufa, sa)
        db = pltpu.async_copy(tb.at[didx.at[k]], bufb, sb)
        return da, db

    def wrt(k, bufa, bufb, sa, sb):
        off = base + k * _CH
        da = pltpu.async_copy(bufa, outa.at[pl.ds(off, _CH)], sa)
        db = pltpu.async_copy(bufb, outb.at[pl.ds(off, _CH)], sb)
        return da, db

    def wait_g(k, bufa, bufb, sa, sb):
        pltpu.make_async_copy(ta.at[sidx.at[k]], bufa, sa).wait()
        pltpu.make_async_copy(tb.at[didx.at[k]], bufb, sb).wait()

    def wait_w(k, bufa, bufb, sa, sb):
        off = base + k * _CH
        pltpu.make_async_copy(bufa, outa.at[pl.ds(off, _CH)], sa).wait()
        pltpu.make_async_copy(bufb, outb.at[pl.ds(off, _CH)], sb).wait()

    gath(0, ba0, bb0, gsa0, gsb0)
    wait_g(0, ba0, bb0, gsa0, gsb0)
    gath(1, ba1, bb1, gsa1, gsb1)
    wrt(0, ba0, bb0, wsa0, wsb0)

    def step(kk, carry):
        k0 = 2 * kk
        wait_g(k0 - 1, ba1, bb1, gsa1, gsb1)
        wait_w(k0 - 2, ba0, bb0, wsa0, wsb0)
        gath(k0, ba0, bb0, gsa0, gsb0)
        wrt(k0 - 1, ba1, bb1, wsa1, wsb1)
        wait_g(k0, ba0, bb0, gsa0, gsb0)
        wait_w(k0 - 1, ba1, bb1, wsa1, wsb1)
        gath(k0 + 1, ba1, bb1, gsa1, gsb1)
        wrt(k0, ba0, bb0, wsa0, wsb0)
        return carry

    lax.fori_loop(1, _KCH_E // 2, step, 0)
    wait_g(_KCH_E - 1, ba1, bb1, gsa1, gsb1)
    wait_w(_KCH_E - 2, ba0, bb0, wsa0, wsb0)
    wrt(_KCH_E - 1, ba1, bb1, wsa1, wsb1)
    wait_w(_KCH_E - 1, ba1, bb1, wsa1, wsb1)


@functools.lru_cache(maxsize=None)
def _egather_kernel():
    return pl.kernel(
        _egather_body,
        out_type=(
            jax.ShapeDtypeStruct((_EUPAD, _DIM), jnp.float32),
            jax.ShapeDtypeStruct((_EUPAD, _DIM), jnp.float32),
        ),
        mesh=_mesh(),
        scratch_types=[
            pltpu.VMEM((_KCH_E, _CH), jnp.int32),
            pltpu.VMEM((_KCH_E, _CH), jnp.int32),
            pltpu.VMEM((_CH, _DIM), jnp.float32),
            pltpu.VMEM((_CH, _DIM), jnp.float32),
            pltpu.VMEM((_CH, _DIM), jnp.float32),
            pltpu.VMEM((_CH, _DIM), jnp.float32),
        ] + [pltpu.SemaphoreType.DMA] * 8,
    )


_BR = 2000


def _mm_scale_body(h_ref, w_ref, degp_ref, out_ref):
    dinv = lax.rsqrt(degp_ref[0, :, 0:1] + degp_ref[1, :, 0:1] + 1.0)
    hw = lax.dot_general(h_ref[...], w_ref[...], (((1,), (1,)), ((), ())),
                         preferred_element_type=jnp.float32)
    out_ref[...] = hw * dinv


def _mm_scale(h, w, degp):
    return pl.pallas_call(
        _mm_scale_body,
        grid=(_N // _BR,),
        in_specs=[
            pl.BlockSpec((_BR, _DIM), lambda i: (i, 0)),
            pl.BlockSpec((_DIM, _DIM), lambda i: (0, 0)),
            pl.BlockSpec((_NC, _BR, _DIM), lambda i: (0, i, 0)),
        ],
        out_specs=pl.BlockSpec((_BR, _DIM), lambda i: (i, 0)),
        out_shape=jax.ShapeDtypeStruct((_N, _DIM), jnp.float32),
    )(h, w, degp)


def _ln_res_body(h_ref, hwp_ref, sp_ref, degp_ref, cb_ref, lw_ref, lb_ref,
                 out_ref):
    dinv = lax.rsqrt(degp_ref[0, :, 0:1] + degp_ref[1, :, 0:1] + 1.0)
    seg = sp_ref[0] + sp_ref[1] + hwp_ref[...]
    u = h_ref[...] + dinv * seg + cb_ref[...]
    mu = jnp.mean(u, axis=-1, keepdims=True)
    d = u - mu
    var = jnp.mean(d * d, axis=-1, keepdims=True)
    out_ref[...] = d * lax.rsqrt(var + 1e-5) * lw_ref[...] + lb_ref[...]


def _ln_res(h, hwp, sp, degp, cb, lw, lb):
    return pl.pallas_call(
        _ln_res_body,
        grid=(_N // _BR,),
        in_specs=[
            pl.BlockSpec((_BR, _DIM), lambda i: (i, 0)),
            pl.BlockSpec((_BR, _DIM), lambda i: (i, 0)),
            pl.BlockSpec((_NC, _BR, _DIM), lambda i: (0, i, 0)),
            pl.BlockSpec((_NC, _BR, _DIM), lambda i: (0, i, 0)),
            pl.BlockSpec((1, _DIM), lambda i: (0, 0)),
            pl.BlockSpec((1, _DIM), lambda i: (0, 0)),
            pl.BlockSpec((1, _DIM), lambda i: (0, 0)),
        ],
        out_specs=pl.BlockSpec((_BR, _DIM), lambda i: (i, 0)),
        out_shape=jax.ShapeDtypeStruct((_N, _DIM), jnp.float32),
    )(h, hwp, sp, degp, cb, lw, lb)


def _ab_body(h_ref, wa_ref, wb_ref, outa_ref, outb_ref):
    h = h_ref[...]
    outa_ref[...] = jnp.dot(h, wa_ref[...], preferred_element_type=jnp.float32)
    outb_ref[...] = jnp.dot(h, wb_ref[...], preferred_element_type=jnp.float32)


def _ab_proj(h, wa, wb):
    return pl.pallas_call(
        _ab_body,
        grid=(_N // _BR,),
        in_specs=[
            pl.BlockSpec((_BR, _DIM), lambda i: (i, 0)),
            pl.BlockSpec((_DIM, _DIM), lambda i: (0, 0)),
            pl.BlockSpec((_DIM, _DIM), lambda i: (0, 0)),
        ],
        out_specs=(
            pl.BlockSpec((_BR, _DIM), lambda i: (i, 0)),
            pl.BlockSpec((_BR, _DIM), lambda i: (i, 0)),
        ),
        out_shape=(
            jax.ShapeDtypeStruct((_N, _DIM), jnp.float32),
            jax.ShapeDtypeStruct((_N, _DIM), jnp.float32),
        ),
    )(h, wa, wb)


_BE = 2000


def _emlp_body(ga_ref, gb_ref, ef_ref, wc_ref, w2_ref, b1_ref, b2_ref,
               out_ref):
    t = (ga_ref[...] + gb_ref[...]
         + jnp.dot(ef_ref[...], wc_ref[...], preferred_element_type=jnp.float32)
         + b1_ref[...])
    hid = jnp.maximum(t, 0.0)
    out_ref[...] = (jnp.dot(hid, w2_ref[...], preferred_element_type=jnp.float32)
                    + b2_ref[...])


def _emlp(ga, gb, ef, wc, w2, b1, b2):
    return pl.pallas_call(
        _emlp_body,
        grid=(_EU // _BE,),
        in_specs=[
            pl.BlockSpec((_BE, _DIM), lambda i: (i, 0)),
            pl.BlockSpec((_BE, _DIM), lambda i: (i, 0)),
            pl.BlockSpec((_BE, 16), lambda i: (i, 0)),
            pl.BlockSpec((16, _DIM), lambda i: (0, 0)),
            pl.BlockSpec((_DIM, _DIM), lambda i: (0, 0)),
            pl.BlockSpec((1, _DIM), lambda i: (0, 0)),
            pl.BlockSpec((1, _DIM), lambda i: (0, 0)),
        ],
        out_specs=pl.BlockSpec((_BE, _DIM), lambda i: (i, 0)),
        out_shape=jax.ShapeDtypeStruct((_EU, _DIM), jnp.float32),
    )(ga, gb, ef, wc, w2, b1, b2)


def kernel(x, edge_index, edge_features, convW, convB, lnW, lnB, eW1, eB1,
           eW2, eB2):
    src_all = edge_index[0]
    dst_all = edge_index[1]
    pad_e = _EPAD - _E
    pad_ar = jnp.arange(pad_e, dtype=jnp.int32)
    src_pad = jnp.concatenate([src_all, pad_ar % _N])
    src_pad = src_pad.reshape(_NW * _KCH, _CH)
    dst_pad = jnp.concatenate([dst_all, _N + pad_ar % (_ACC - _N)])
    dst_pad = dst_pad.reshape(_NW * _KCH, _CH)

    zeros128 = jnp.zeros((_ACC, _DIM), jnp.float32)
    ones128 = jnp.ones((_CH, _DIM), jnp.float32)

    degp = _deg_kernel()(dst_pad, ones128, zeros128)

    h = x
    for l in range(_NLAYERS):
        hwp = _mm_scale(h, convW[l], degp)
        sp = _agg_kernel()(hwp, src_pad, dst_pad, zeros128)
        h = _ln_res(h, hwp, sp, degp, convB[l].reshape(1, _DIM),
                    lnW[l].reshape(1, _DIM), lnB[l].reshape(1, _DIM))

    srcu = edge_index[0, 0::2]
    dstu = edge_index[1, 0::2]
    pad_u = _EUPAD - _EU
    pad_au = jnp.arange(pad_u, dtype=jnp.int32) % _N
    srcu_pad = jnp.concatenate([srcu, pad_au])
    srcu_pad = srcu_pad.reshape(_NW * _KCH_E, _CH)
    dstu_pad = jnp.concatenate([dstu, pad_au])
    dstu_pad = dstu_pad.reshape(_NW * _KCH_E, _CH)

    e_w1t = eW1.T
    wa = e_w1t[:_DIM]
    wb = e_w1t[_DIM:2 * _DIM]
    wc = e_w1t[2 * _DIM:]
    a, b = _ab_proj(h, wa, wb)
    ga, gb = _egather_kernel()(a, b, srcu_pad, dstu_pad)
    edge_emb = _emlp(ga, gb, edge_features, wc, eW2.T,
                     eB1.reshape(1, _DIM), eB2.reshape(1, _DIM))
    return (h, edge_emb)


# speedup vs baseline: 2.6556x; 1.0503x over previous
import functools

import jax
import jax.numpy as jnp
from jax import lax
from jax.experimental import pallas as pl
from jax.experimental.pallas import tpu as pltpu
from jax.experimental.pallas import tpu_sc as plsc

_N = 10000
_DIM = 128
_E = 320000
_EU = 160000
_NLAYERS = 3
_NC, _NS, _NW = 2, 16, 32
_CH = 128
_KCH = 80
_EPAD = _NW * _KCH * _CH
_KCH_E = 40
_EUPAD = _NW * _KCH_E * _CH
_ACC = 10112
_STRIPE = _ACC // _NS

@functools.lru_cache(maxsize=None)
def _mesh():
    return plsc.VectorSubcoreMesh(core_axis_name="c", subcore_axis_name="s")


def _deg_body(dst_i, ones_h, zeros_h, out, acc, ones_v, didx, ss0, ss1):
    c = lax.axis_index("c")
    s = lax.axis_index("s")
    wid = c * _NS + s
    row0 = s * _STRIPE
    pltpu.sync_copy(zeros_h.at[pl.ds(row0, _STRIPE)], acc.at[pl.ds(row0, _STRIPE)])
    pltpu.sync_copy(ones_h, ones_v)
    pltpu.sync_copy(dst_i.at[pl.ds(wid * _KCH, _KCH)], didx)
    plsc.subcore_barrier()

    def scatter(k, sem):
        return pltpu.async_copy(ones_v, acc.at[didx.at[k]], sem, add=True)

    def wait_s(k, sem):
        pltpu.make_async_copy(ones_v, acc.at[didx.at[k]], sem).wait()

    scatter(0, ss0)
    scatter(1, ss1)

    def step(kk, carry):
        k0 = 2 * kk
        wait_s(k0 - 2, ss0)
        scatter(k0, ss0)
        wait_s(k0 - 1, ss1)
        scatter(k0 + 1, ss1)
        return carry

    lax.fori_loop(1, _KCH // 2, step, 0)
    wait_s(_KCH - 2, ss0)
    wait_s(_KCH - 1, ss1)
    plsc.subcore_barrier()
    pltpu.sync_copy(acc.at[pl.ds(row0, _STRIPE)], out.at[c, pl.ds(row0, _STRIPE)])


@functools.lru_cache(maxsize=None)
def _deg_kernel():
    return pl.kernel(
        _deg_body,
        out_type=jax.ShapeDtypeStruct((_NC, _ACC, _DIM), jnp.float32),
        mesh=_mesh(),
        scratch_types=[
            pltpu.VMEM_SHARED((_ACC, _DIM), jnp.float32),
            pltpu.VMEM((_CH, _DIM), jnp.float32),
            pltpu.VMEM((_KCH, _CH), jnp.int32),
            pltpu.SemaphoreType.DMA,
            pltpu.SemaphoreType.DMA,
        ],
    )


def _agg_body(table, src_i, dst_i, zeros_h, out, acc, sidx, didx,
              rows0, rows1, gs0, gs1, ss0, ss1):
    c = lax.axis_index("c")
    s = lax.axis_index("s")
    wid = c * _NS + s
    row0 = s * _STRIPE
    pltpu.sync_copy(zeros_h.at[pl.ds(row0, _STRIPE)], acc.at[pl.ds(row0, _STRIPE)])
    plsc.subcore_barrier()

    def gather(k, buf, sem):
        return pltpu.async_copy(table.at[sidx.at[k]], buf, sem)

    def scatter(k, buf, sem):
        return pltpu.async_copy(buf, acc.at[didx.at[k]], sem, add=True)

    kh = _KCH // 2
    for half in range(2):
        hb = wid * _KCH + half * kh
        pltpu.sync_copy(src_i.at[pl.ds(hb, kh)], sidx)
        pltpu.sync_copy(dst_i.at[pl.ds(hb, kh)], didx)
        gather(0, rows0, gs0).wait()
        gather(1, rows1, gs1)
        scatter(0, rows0, ss0)

        def step(kk, carry):
            k0 = 2 * kk
            pltpu.make_async_copy(table.at[sidx.at[k0 - 1]], rows1, gs1).wait()
            pltpu.make_async_copy(rows0, acc.at[didx.at[k0 - 2]], ss0).wait()
            gather(k0, rows0, gs0)
            scatter(k0 - 1, rows1, ss1)
            pltpu.make_async_copy(table.at[sidx.at[k0]], rows0, gs0).wait()
            pltpu.make_async_copy(rows1, acc.at[didx.at[k0 - 1]], ss1).wait()
            gather(k0 + 1, rows1, gs1)
            scatter(k0, rows0, ss0)
            return carry

        lax.fori_loop(1, kh // 2, step, 0)
        pltpu.make_async_copy(table.at[sidx.at[kh - 1]], rows1, gs1).wait()
        pltpu.make_async_copy(rows0, acc.at[didx.at[kh - 2]], ss0).wait()
        scatter(kh - 1, rows1, ss1).wait()
    plsc.subcore_barrier()
    pltpu.sync_copy(acc.at[pl.ds(row0, _STRIPE)], out.at[c, pl.ds(row0, _STRIPE)])


@functools.lru_cache(maxsize=None)
def _agg_kernel():
    return pl.kernel(
        _agg_body,
        out_type=jax.ShapeDtypeStruct((_NC, _ACC, _DIM), jnp.float32),
        mesh=_mesh(),
        scratch_types=[
            pltpu.VMEM_SHARED((_ACC, _DIM), jnp.float32),
            pltpu.VMEM((_KCH // 2, _CH), jnp.int32),
            pltpu.VMEM((_KCH // 2, _CH), jnp.int32),
            pltpu.VMEM((_CH, _DIM), jnp.float32),
            pltpu.VMEM((_CH, _DIM), jnp.float32),
            pltpu.SemaphoreType.DMA,
            pltpu.SemaphoreType.DMA,
            pltpu.SemaphoreType.DMA,
            pltpu.SemaphoreType.DMA,
        ],
    )


def _egather_body(ta, tb, src_i, dst_i, out, sidx, didx, b0, b1,
                  gsa0, gsb0, ws0, gsa1, gsb1, ws1):
    c = lax.axis_index("c")
    s = lax.axis_index("s")
    wid = c * _NS + s
    base = wid * _KCH_E * _CH
    pltpu.sync_copy(src_i.at[pl.ds(wid * _KCH_E, _KCH_E)], sidx)
    pltpu.sync_copy(dst_i.at[pl.ds(wid * _KCH_E, _KCH_E)], didx)

    def ga(k, buf, sem):
        return pltpu.async_copy(ta.at[sidx.at[k]], buf, sem)

    def gb(k, buf, sem):
        return pltpu.async_copy(tb.at[didx.at[k]], buf, sem, add=True)

    def wrt(k, buf, sem):
        return pltpu.async_copy(buf, out.at[pl.ds(base + k * _CH, _CH)], sem)

    def wait_ga(k, buf, sem):
        pltpu.make_async_copy(ta.at[sidx.at[k]], buf, sem).wait()

    def wait_gb(k, buf, sem):
        pltpu.make_async_copy(tb.at[didx.at[k]], buf, sem).wait()

    def wait_w(k, buf, sem):
        pltpu.make_async_copy(buf, out.at[pl.ds(base + k * _CH, _CH)], sem).wait()

    ga(0, b0, gsa0)
    wait_ga(0, b0, gsa0)
    gb(0, b0, gsb0)
    ga(1, b1, gsa1)

    def step(kk, carry):
        k0 = 2 * kk
        wait_gb(k0 - 2, b0, gsb0)
        wrt(k0 - 2, b0, ws0)
        wait_ga(k0 - 1, b1, gsa1)
        gb(k0 - 1, b1, gsb1)
        wait_w(k0 - 2, b0, ws0)
        ga(k0, b0, gsa0)
        wait_gb(k0 - 1, b1, gsb1)
        wrt(k0 - 1, b1, ws1)
        wait_ga(k0, b0, gsa0)
        gb(k0, b0, gsb0)
        wait_w(k0 - 1, b1, ws1)
        ga(k0 + 1, b1, gsa1)
        return carry

    lax.fori_loop(1, _KCH_E // 2, step, 0)
    wait_gb(_KCH_E - 2, b0, gsb0)
    wrt(_KCH_E - 2, b0, ws0)
    wait_ga(_KCH_E - 1, b1, gsa1)
    gb(_KCH_E - 1, b1, gsb1)
    wait_gb(_KCH_E - 1, b1, gsb1)
    wrt(_KCH_E - 1, b1, ws1)
    wait_w(_KCH_E - 2, b0, ws0)
    wait_w(_KCH_E - 1, b1, ws1)


@functools.lru_cache(maxsize=None)
def _egather_kernel():
    return pl.kernel(
        _egather_body,
        out_type=jax.ShapeDtypeStruct((_EUPAD, _DIM), jnp.float32),
        mesh=_mesh(),
        scratch_types=[
            pltpu.VMEM((_KCH_E, _CH), jnp.int32),
            pltpu.VMEM((_KCH_E, _CH), jnp.int32),
            pltpu.VMEM((_CH, _DIM), jnp.float32),
            pltpu.VMEM((_CH, _DIM), jnp.float32),
        ] + [pltpu.SemaphoreType.DMA] * 6,
    )


_BR = 2000


def _mm_body(h_ref, w_ref, out_ref):
    out_ref[...] = lax.dot_general(h_ref[...], w_ref[...],
                                   (((1,), (1,)), ((), ())),
                                   preferred_element_type=jnp.float32)


def _mm_plain(h, w):
    return pl.pallas_call(
        _mm_body,
        grid=(_N // _BR,),
        in_specs=[
            pl.BlockSpec((_BR, _DIM), lambda i: (i, 0)),
            pl.BlockSpec((_DIM, _DIM), lambda i: (0, 0)),
        ],
        out_specs=pl.BlockSpec((_BR, _DIM), lambda i: (i, 0)),
        out_shape=jax.ShapeDtypeStruct((_N, _DIM), jnp.float32),
    )(h, w)


def _scale_body(hw_ref, degp_ref, out_ref):
    dinv = lax.rsqrt(degp_ref[0, :, 0:1] + degp_ref[1, :, 0:1] + 1.0)
    out_ref[...] = hw_ref[...] * dinv


def _scale(hw, degp):
    return pl.pallas_call(
        _scale_body,
        grid=(_N // _BR,),
        in_specs=[
            pl.BlockSpec((_BR, _DIM), lambda i: (i, 0)),
            pl.BlockSpec((_NC, _BR, _DIM), lambda i: (0, i, 0)),
        ],
        out_specs=pl.BlockSpec((_BR, _DIM), lambda i: (i, 0)),
        out_shape=jax.ShapeDtypeStruct((_N, _DIM), jnp.float32),
    )(hw, degp)


def _mm_scale_body(h_ref, w_ref, degp_ref, out_ref):
    dinv = lax.rsqrt(degp_ref[0, :, 0:1] + degp_ref[1, :, 0:1] + 1.0)
    hw = lax.dot_general(h_ref[...], w_ref[...], (((1,), (1,)), ((), ())),
                         preferred_element_type=jnp.float32)
    out_ref[...] = hw * dinv


def _mm_scale(h, w, degp):
    return pl.pallas_call(
        _mm_scale_body,
        grid=(_N // _BR,),
        in_specs=[
            pl.BlockSpec((_BR, _DIM), lambda i: (i, 0)),
            pl.BlockSpec((_DIM, _DIM), lambda i: (0, 0)),
            pl.BlockSpec((_NC, _BR, _DIM), lambda i: (0, i, 0)),
        ],
        out_specs=pl.BlockSpec((_BR, _DIM), lambda i: (i, 0)),
        out_shape=jax.ShapeDtypeStruct((_N, _DIM), jnp.float32),
    )(h, w, degp)


def _ln_res_body(h_ref, hwp_ref, sp_ref, degp_ref, cb_ref, lw_ref, lb_ref,
                 out_ref):
    dinv = lax.rsqrt(degp_ref[0, :, 0:1] + degp_ref[1, :, 0:1] + 1.0)
    seg = sp_ref[0] + sp_ref[1] + hwp_ref[...]
    u = h_ref[...] + dinv * seg + cb_ref[...]
    mu = jnp.mean(u, axis=-1, keepdims=True)
    d = u - mu
    var = jnp.mean(d * d, axis=-1, keepdims=True)
    out_ref[...] = d * lax.rsqrt(var + 1e-5) * lw_ref[...] + lb_ref[...]


def _ln_res(h, hwp, sp, degp, cb, lw, lb):
    return pl.pallas_call(
        _ln_res_body,
        grid=(_N // _BR,),
        in_specs=[
            pl.BlockSpec((_BR, _DIM), lambda i: (i, 0)),
            pl.BlockSpec((_BR, _DIM), lambda i: (i, 0)),
            pl.BlockSpec((_NC, _BR, _DIM), lambda i: (0, i, 0)),
            pl.BlockSpec((_NC, _BR, _DIM), lambda i: (0, i, 0)),
            pl.BlockSpec((1, _DIM), lambda i: (0, 0)),
            pl.BlockSpec((1, _DIM), lambda i: (0, 0)),
            pl.BlockSpec((1, _DIM), lambda i: (0, 0)),
        ],
        out_specs=pl.BlockSpec((_BR, _DIM), lambda i: (i, 0)),
        out_shape=jax.ShapeDtypeStruct((_N, _DIM), jnp.float32),
    )(h, hwp, sp, degp, cb, lw, lb)


def _ab_body(h_ref, wa_ref, wb_ref, outa_ref, outb_ref):
    h = h_ref[...]
    outa_ref[...] = jnp.dot(h, wa_ref[...], preferred_element_type=jnp.float32)
    outb_ref[...] = jnp.dot(h, wb_ref[...], preferred_element_type=jnp.float32)


def _ab_proj(h, wa, wb):
    return pl.pallas_call(
        _ab_body,
        grid=(_N // _BR,),
        in_specs=[
            pl.BlockSpec((_BR, _DIM), lambda i: (i, 0)),
            pl.BlockSpec((_DIM, _DIM), lambda i: (0, 0)),
            pl.BlockSpec((_DIM, _DIM), lambda i: (0, 0)),
        ],
        out_specs=(
            pl.BlockSpec((_BR, _DIM), lambda i: (i, 0)),
            pl.BlockSpec((_BR, _DIM), lambda i: (i, 0)),
        ),
        out_shape=(
            jax.ShapeDtypeStruct((_N, _DIM), jnp.float32),
            jax.ShapeDtypeStruct((_N, _DIM), jnp.float32),
        ),
    )(h, wa, wb)


_BE = 2000


def _emlp_body(g_ref, ef_ref, wc_ref, w2_ref, b1_ref, b2_ref, out_ref):
    t = (g_ref[...]
         + jnp.dot(ef_ref[...], wc_ref[...], preferred_element_type=jnp.float32)
         + b1_ref[...])
    hid = jnp.maximum(t, 0.0)
    out_ref[...] = (jnp.dot(hid, w2_ref[...], preferred_element_type=jnp.float32)
                    + b2_ref[...])


def _emlp(g, ef, wc, w2, b1, b2):
    return pl.pallas_call(
        _emlp_body,
        grid=(_EU // _BE,),
        in_specs=[
            pl.BlockSpec((_BE, _DIM), lambda i: (i, 0)),
            pl.BlockSpec((_BE, 16), lambda i: (i, 0)),
            pl.BlockSpec((16, _DIM), lambda i: (0, 0)),
            pl.BlockSpec((_DIM, _DIM), lambda i: (0, 0)),
            pl.BlockSpec((1, _DIM), lambda i: (0, 0)),
            pl.BlockSpec((1, _DIM), lambda i: (0, 0)),
        ],
        out_specs=pl.BlockSpec((_BE, _DIM), lambda i: (i, 0)),
        out_shape=jax.ShapeDtypeStruct((_EU, _DIM), jnp.float32),
    )(g, ef, wc, w2, b1, b2)


def kernel(x, edge_index, edge_features, convW, convB, lnW, lnB, eW1, eB1,
           eW2, eB2):
    src_all = edge_index[0]
    dst_all = edge_index[1]
    pad_e = _EPAD - _E
    pad_ar = jnp.arange(pad_e, dtype=jnp.int32)
    src_pad = jnp.concatenate([src_all, pad_ar % _N])
    src_pad = src_pad.reshape(_NW * _KCH, _CH)
    dst_pad = jnp.concatenate([dst_all, _N + pad_ar % (_ACC - _N)])
    dst_pad = dst_pad.reshape(_NW * _KCH, _CH)

    zeros128 = jnp.zeros((_ACC, _DIM), jnp.float32)
    ones128 = jnp.ones((_CH, _DIM), jnp.float32)

    degp = _deg_kernel()(dst_pad, ones128, zeros128)

    h = x
    for l in range(_NLAYERS):
        if l == 0:
            hwp = _scale(_mm_plain(h, convW[l]), degp)
        else:
            hwp = _mm_scale(h, convW[l], degp)
        sp = _agg_kernel()(hwp, src_pad, dst_pad, zeros128)
        h = _ln_res(h, hwp, sp, degp, convB[l].reshape(1, _DIM),
                    lnW[l].reshape(1, _DIM), lnB[l].reshape(1, _DIM))

    srcu = edge_index[0, 0::2]
    dstu = edge_index[1, 0::2]
    pad_u = _EUPAD - _EU
    pad_au = jnp.arange(pad_u, dtype=jnp.int32) % _N
    srcu_pad = jnp.concatenate([srcu, pad_au])
    srcu_pad = srcu_pad.reshape(_NW * _KCH_E, _CH)
    dstu_pad = jnp.concatenate([dstu, pad_au])
    dstu_pad = dstu_pad.reshape(_NW * _KCH_E, _CH)

    e_w1t = eW1.T
    wa = e_w1t[:_DIM]
    wb = e_w1t[_DIM:2 * _DIM]
    wc = e_w1t[2 * _DIM:]
    a, b = _ab_proj(h, wa, wb)
    g = _egather_kernel()(a, b, srcu_pad, dstu_pad)
    edge_emb = _emlp(g, edge_features, wc, eW2.T,
                     eB1.reshape(1, _DIM), eB2.reshape(1, _DIM))
    return (h, edge_emb)
